# baseline (speedup 1.0000x reference)
import functools

import jax
import jax.numpy as jnp
from jax.experimental import pallas as pl
from jax.experimental.pallas import tpu as pltpu

_NEG = -1e30


def _shift(a, d):
    if d == 0:
        return a
    return jnp.concatenate([a[d:], a[:d]], axis=0)


def _gat_layer(h, asrc, adst, pens, grpshift, heads, ch):
    exs = {}
    for c in range(4):
        s = grpshift(asrc, c) + adst
        e = jnp.where(s > 0, s, 0.2 * s)
        exs[c] = jnp.exp(e if c == 0 else e + pens[c])

    denom = functools.reduce(jnp.add, exs.values()) + 1e-16
    inv = 1.0 / denom

    if heads > 1:
        rep = (jax.lax.broadcasted_iota(jnp.int32, (heads, heads * ch), 1)
               // ch == jax.lax.broadcasted_iota(
                   jnp.int32, (heads, heads * ch), 0)).astype(jnp.bfloat16)
    out = None
    for c in range(4):
        coef = (exs[c] * inv).astype(jnp.bfloat16)
        if heads > 1:
            coef = jnp.dot(coef, rep,
                           preferred_element_type=jnp.float32).astype(jnp.bfloat16)
        contrib = coef * grpshift(h, c)
        out = contrib if out is None else out + contrib
    return out


def _fused_kernel(x_ref, mp_ref, wv_ref, bv_ref, w1_ref,
                  b1_ref, w2_ref, b2_ref, fw_ref, fb_ref,
                  o_ref, *, heads, ch1, ch2):
    tr = x_ref.shape[1]
    c1 = heads * ch1
    bf = jnp.bfloat16
    x = x_ref[0]
    z = jnp.dot(x, wv_ref[0], preferred_element_type=jnp.float32) + bv_ref[0]
    h1a = jnp.dot(z.astype(bf), w1_ref[...],
                  preferred_element_type=jnp.float32).astype(bf)
    h1 = h1a[:, :c1]
    asrc1 = h1a[:, c1:c1 + heads].astype(jnp.float32)
    adst1 = h1a[:, c1 + heads:c1 + 2 * heads].astype(jnp.float32)

    mp = mp_ref[0]
    kio = jax.lax.broadcasted_iota(jnp.int32, (tr, 1), 0) & 3
    own_bit = (mp >> kio) & 1

    conds = {c: kio < (4 - c) for c in (1, 2, 3)}

    def grpshift(a, c):
        if c == 0:
            return a
        return jnp.where(conds[c], _shift(a, c), _shift(a, c - 4))

    pens = {c: jnp.where((own_bit & grpshift(own_bit, c)) == 1, 0.0, _NEG)
            for c in (1, 2, 3)}

    out1 = _gat_layer(h1, asrc1, adst1, pens, grpshift, heads, ch1)
    hmid = jnp.maximum(out1 + b1_ref[...], jnp.bfloat16(0.0))
    h2a = jnp.dot(hmid, w2_ref[...],
                  preferred_element_type=jnp.float32).astype(bf)
    h2 = h2a[:, :ch2]
    asrc2 = h2a[:, ch2:ch2 + 1].astype(jnp.float32)
    adst2 = h2a[:, ch2 + 1:ch2 + 2].astype(jnp.float32)
    out2 = _gat_layer(h2, asrc2, adst2, pens, grpshift, 1, ch2)
    hf = jnp.maximum(out2 + b2_ref[...], jnp.bfloat16(0.0))
    o_ref[...] = (jnp.dot(hf, fw_ref[...],
                          preferred_element_type=jnp.float32) + fb_ref[...])


def _pick_tile(b):
    best = 8
    for t in range(8, min(b, 1024) + 1, 8):
        if b % t == 0 and t % 4 == 0:
            best = t
    return best


def kernel(X, missing_pattern, view_W, view_b, W1, att_src1, att_dst1, b1,
           W2, att_src2, att_dst2, b2, fc_W, fc_b):
    V, B, in_dim = X.shape
    d_model = view_W.shape[2]
    heads, ch1 = att_src1.shape
    ch2 = att_src2.shape[1]
    out_dim = fc_W.shape[1]
    TR = _pick_tile(B)

    mpn = jnp.repeat(missing_pattern.astype(jnp.int32), 4).reshape(V, B, 1)
    bv = view_b.reshape(V, 1, d_model)
    fbr = fc_b.reshape(1, out_dim)

    def att_matrix(a_s, a_d):
        nh, c = a_s.shape
        eye = jnp.eye(nh, dtype=jnp.float32)
        left = (a_s[:, :, None] * eye[:, None, :]).reshape(nh * c, nh)
        right = (a_d[:, :, None] * eye[:, None, :]).reshape(nh * c, nh)
        return jnp.concatenate([left, right], axis=1)

    bf = jnp.bfloat16
    xb = X.astype(bf)
    wvb = view_W.astype(bf)
    w1aug = jnp.concatenate(
        [W1, W1 @ att_matrix(att_src1, att_dst1)], axis=1).astype(bf)
    w2aug = jnp.concatenate(
        [W2, W2 @ att_matrix(att_src2, att_dst2)], axis=1).astype(bf)
    fwb = fc_W.astype(bf)
    b1r = b1.reshape(1, heads * ch1).astype(bf)
    b2r = b2.reshape(1, ch2).astype(bf)

    grid = (V, B // TR)
    fixed = lambda v, c: (0, 0)
    out2d = pl.pallas_call(
        functools.partial(_fused_kernel, heads=heads, ch1=ch1, ch2=ch2),
        grid=grid,
        in_specs=[
            pl.BlockSpec((1, TR, in_dim), lambda v, c: (v, c, 0)),
            pl.BlockSpec((1, TR, 1), lambda v, c: (v, c, 0)),
            pl.BlockSpec((1, in_dim, d_model), lambda v, c: (v, 0, 0)),
            pl.BlockSpec((1, 1, d_model), lambda v, c: (v, 0, 0)),
            pl.BlockSpec(w1aug.shape, fixed),
            pl.BlockSpec(b1r.shape, fixed),
            pl.BlockSpec(w2aug.shape, fixed),
            pl.BlockSpec(b2r.shape, fixed),
            pl.BlockSpec(fwb.shape, fixed),
            pl.BlockSpec(fbr.shape, fixed),
        ],
        out_specs=pl.BlockSpec((TR, out_dim), lambda v, c: (c, v)),
        out_shape=jax.ShapeDtypeStruct((B, V * out_dim), jnp.float32),
    )(xb, mpn, wvb, bv, w1aug, b1r, w2aug, b2r, fwb, fbr)
    return out2d.reshape(B, V, out_dim)


# speedup vs baseline: 1.1015x; 1.1015x over previous
import functools

import jax
import jax.numpy as jnp
from jax.experimental import pallas as pl
from jax.experimental.pallas import tpu as pltpu

_NEG = -1e30


def _shift(a, d):
    if d == 0:
        return a
    return jnp.concatenate([a[d:], a[:d]], axis=0)


def _gat_layer(h, asrc, adst, pens, grpshift, heads, ch):
    exs = {}
    for c in range(4):
        s = grpshift(asrc, c) + adst
        e = jnp.where(s > 0, s, 0.2 * s)
        exs[c] = jnp.exp(e if c == 0 else e + pens[c])

    denom = functools.reduce(jnp.add, exs.values()) + 1e-16
    inv = 1.0 / denom

    if heads > 1:
        rep = (jax.lax.broadcasted_iota(jnp.int32, (heads, heads * ch), 1)
               // ch == jax.lax.broadcasted_iota(
                   jnp.int32, (heads, heads * ch), 0)).astype(jnp.float32)
    out = None
    for c in range(4):
        coef = exs[c] * inv
        if heads > 1:
            coef = jnp.dot(coef, rep, preferred_element_type=jnp.float32)
        contrib = coef * grpshift(h, c)
        out = contrib if out is None else out + contrib
    return out


def _fused_kernel(x_ref, mp_ref, wv_ref, bv_ref, w1_ref,
                  b1_ref, w2_ref, b2_ref, fw_ref, fb_ref,
                  o_ref, *, heads, ch1, ch2):
    tr = x_ref.shape[1]
    c1 = heads * ch1
    bf = jnp.bfloat16
    x = x_ref[0]
    z = jnp.dot(x, wv_ref[0], preferred_element_type=jnp.float32) + bv_ref[0]
    h1a = jnp.dot(z.astype(bf), w1_ref[...],
                  preferred_element_type=jnp.float32)
    h1 = h1a[:, :c1]
    asrc1 = h1a[:, c1:c1 + heads]
    adst1 = h1a[:, c1 + heads:c1 + 2 * heads]

    mp = mp_ref[0]
    kio = jax.lax.broadcasted_iota(jnp.int32, (tr, 1), 0) & 3
    own_bit = (mp >> kio) & 1

    conds = {c: kio < (4 - c) for c in (1, 2, 3)}

    def grpshift(a, c):
        if c == 0:
            return a
        return jnp.where(conds[c], _shift(a, c), _shift(a, c - 4))

    pens = {c: jnp.where((own_bit & grpshift(own_bit, c)) == 1, 0.0, _NEG)
            for c in (1, 2, 3)}

    out1 = _gat_layer(h1, asrc1, adst1, pens, grpshift, heads, ch1)
    hmid = jnp.maximum(out1 + b1_ref[...], 0.0)
    h2a = jnp.dot(hmid.astype(bf), w2_ref[...],
                  preferred_element_type=jnp.float32)
    h2 = h2a[:, :ch2]
    asrc2 = h2a[:, ch2:ch2 + 1]
    adst2 = h2a[:, ch2 + 1:ch2 + 2]
    out2 = _gat_layer(h2, asrc2, adst2, pens, grpshift, 1, ch2)
    hf = jnp.maximum(out2 + b2_ref[...], 0.0)
    o_ref[...] = (jnp.dot(hf.astype(bf), fw_ref[...],
                          preferred_element_type=jnp.float32) + fb_ref[...])


def _pick_tile(b):
    best = 8
    for t in range(8, min(b, 5000) + 1, 8):
        if b % t == 0 and t % 4 == 0:
            best = t
    return best


def kernel(X, missing_pattern, view_W, view_b, W1, att_src1, att_dst1, b1,
           W2, att_src2, att_dst2, b2, fc_W, fc_b):
    V, B, in_dim = X.shape
    d_model = view_W.shape[2]
    heads, ch1 = att_src1.shape
    ch2 = att_src2.shape[1]
    out_dim = fc_W.shape[1]
    TR = _pick_tile(B)

    mpn = jnp.repeat(missing_pattern.astype(jnp.int32), 4).reshape(V, B, 1)
    bv = view_b.reshape(V, 1, d_model)
    fbr = fc_b.reshape(1, out_dim)

    def att_matrix(a_s, a_d):
        nh, c = a_s.shape
        eye = jnp.eye(nh, dtype=jnp.float32)
        left = (a_s[:, :, None] * eye[:, None, :]).reshape(nh * c, nh)
        right = (a_d[:, :, None] * eye[:, None, :]).reshape(nh * c, nh)
        return jnp.concatenate([left, right], axis=1)

    bf = jnp.bfloat16
    xb = X.astype(bf)
    wvb = view_W.astype(bf)
    w1aug = jnp.concatenate(
        [W1, W1 @ att_matrix(att_src1, att_dst1)], axis=1).astype(bf)
    w2aug = jnp.concatenate(
        [W2, W2 @ att_matrix(att_src2, att_dst2)], axis=1).astype(bf)
    fwb = fc_W.astype(bf)
    b1r = b1.reshape(1, heads * ch1)
    b2r = b2.reshape(1, ch2)

    grid = (V, B // TR)
    fixed = lambda v, c: (0, 0)
    out2d = pl.pallas_call(
        functools.partial(_fused_kernel, heads=heads, ch1=ch1, ch2=ch2),
        grid=grid,
        in_specs=[
            pl.BlockSpec((1, TR, in_dim), lambda v, c: (v, c, 0)),
            pl.BlockSpec((1, TR, 1), lambda v, c: (v, c, 0)),
            pl.BlockSpec((1, in_dim, d_model), lambda v, c: (v, 0, 0)),
            pl.BlockSpec((1, 1, d_model), lambda v, c: (v, 0, 0)),
            pl.BlockSpec(w1aug.shape, fixed),
            pl.BlockSpec(b1r.shape, fixed),
            pl.BlockSpec(w2aug.shape, fixed),
            pl.BlockSpec(b2r.shape, fixed),
            pl.BlockSpec(fwb.shape, fixed),
            pl.BlockSpec(fbr.shape, fixed),
        ],
        out_specs=pl.BlockSpec((TR, out_dim), lambda v, c: (c, v)),
        out_shape=jax.ShapeDtypeStruct((B, V * out_dim), jnp.float32),
    )(xb, mpn, wvb, bv, w1aug, b1r, w2aug, b2r, fwb, fbr)
    return out2d.reshape(B, V, out_dim)


# speedup vs baseline: 1.1782x; 1.0695x over previous
import functools

import jax
import jax.numpy as jnp
from jax.experimental import pallas as pl
from jax.experimental.pallas import tpu as pltpu

_NEG = -1e30


def _shift(a, d):
    if d == 0:
        return a
    return jnp.concatenate([a[d:], a[:d]], axis=0)


def _gat_layer(h, asrc, adst, pens, grpshift, heads, ch):
    exs = {}
    for c in range(4):
        s = grpshift(asrc, c) + adst
        e = jnp.where(s > 0, s, 0.2 * s)
        exs[c] = jnp.exp(e if c == 0 else e + pens[c])

    denom = functools.reduce(jnp.add, exs.values()) + 1e-16
    inv = 1.0 / denom

    if heads > 1:
        rep = (jax.lax.broadcasted_iota(jnp.int32, (heads, heads * ch), 1)
               // ch == jax.lax.broadcasted_iota(
                   jnp.int32, (heads, heads * ch), 0)).astype(jnp.float32)
    out = None
    for c in range(4):
        coef = exs[c] * inv
        if heads > 1:
            coef = jnp.dot(coef, rep, preferred_element_type=jnp.float32)
        contrib = coef * grpshift(h, c)
        out = contrib if out is None else out + contrib
    return out


def _fused_kernel(x_ref, mp_ref, wv_ref, bv_ref, w1_ref,
                  b1_ref, w2_ref, b2_ref, fw_ref, fb_ref,
                  o_ref, *, heads, ch1, ch2):
    tr = x_ref.shape[1]
    c1 = heads * ch1
    bf = jnp.bfloat16
    x = x_ref[0]
    z = jnp.dot(x, wv_ref[0], preferred_element_type=jnp.float32) + bv_ref[0]
    h1a = jnp.dot(z.astype(bf), w1_ref[...],
                  preferred_element_type=jnp.float32)
    h1 = h1a[:, :c1]
    asrc1 = h1a[:, c1:c1 + heads]
    adst1 = h1a[:, c1 + heads:c1 + 2 * heads]

    mp = mp_ref[0]
    kio = jax.lax.broadcasted_iota(jnp.int32, (tr, 1), 0) & 3
    own_bit = (mp >> kio) & 1

    conds = {c: kio < (4 - c) for c in (1, 2, 3)}

    def grpshift(a, c):
        if c == 0:
            return a
        return jnp.where(conds[c], _shift(a, c), _shift(a, c - 4))

    pens = {c: jnp.where((own_bit & grpshift(own_bit, c)) == 1, 0.0, _NEG)
            for c in (1, 2, 3)}

    out1 = _gat_layer(h1, asrc1, adst1, pens, grpshift, heads, ch1)
    hmid = jnp.maximum(out1 + b1_ref[...], 0.0)
    h2a = jnp.dot(hmid.astype(bf), w2_ref[...],
                  preferred_element_type=jnp.float32)
    h2 = h2a[:, :ch2]
    asrc2 = h2a[:, ch2:ch2 + 1]
    adst2 = h2a[:, ch2 + 1:ch2 + 2]
    out2 = _gat_layer(h2, asrc2, adst2, pens, grpshift, 1, ch2)
    hf = jnp.maximum(out2 + b2_ref[...], 0.0)
    o_ref[...] = (jnp.dot(hf.astype(bf), fw_ref[...],
                          preferred_element_type=jnp.float32) + fb_ref[...])


def _pick_tile(b):
    best = 8
    for t in range(8, min(b, 5000) + 1, 8):
        if b % t == 0 and t % 4 == 0:
            best = t
    return best


def kernel(X, missing_pattern, view_W, view_b, W1, att_src1, att_dst1, b1,
           W2, att_src2, att_dst2, b2, fc_W, fc_b):
    V, B, in_dim = X.shape
    d_model = view_W.shape[2]
    heads, ch1 = att_src1.shape
    ch2 = att_src2.shape[1]
    out_dim = fc_W.shape[1]
    TR = _pick_tile(B)

    mpn = jnp.repeat(missing_pattern.astype(jnp.int32), 4).reshape(V, B, 1)
    bv = view_b.reshape(V, 1, d_model)
    fbr = fc_b.reshape(1, out_dim)

    def att_matrix(a_s, a_d):
        nh, c = a_s.shape
        eye = jnp.eye(nh, dtype=jnp.float32)
        left = (a_s[:, :, None] * eye[:, None, :]).reshape(nh * c, nh)
        right = (a_d[:, :, None] * eye[:, None, :]).reshape(nh * c, nh)
        return jnp.concatenate([left, right], axis=1)

    bf = jnp.bfloat16
    w1aug = jnp.concatenate(
        [W1, W1 @ att_matrix(att_src1, att_dst1)], axis=1).astype(bf)
    w2aug = jnp.concatenate(
        [W2, W2 @ att_matrix(att_src2, att_dst2)], axis=1).astype(bf)
    fwb = fc_W.astype(bf)
    b1r = b1.reshape(1, heads * ch1)
    b2r = b2.reshape(1, ch2)

    grid = (V, B // TR)
    fixed = lambda v, c: (0, 0)
    out2d = pl.pallas_call(
        functools.partial(_fused_kernel, heads=heads, ch1=ch1, ch2=ch2),
        grid=grid,
        in_specs=[
            pl.BlockSpec((1, TR, in_dim), lambda v, c: (v, c, 0)),
            pl.BlockSpec((1, TR, 1), lambda v, c: (v, c, 0)),
            pl.BlockSpec((1, in_dim, d_model), lambda v, c: (v, 0, 0)),
            pl.BlockSpec((1, 1, d_model), lambda v, c: (v, 0, 0)),
            pl.BlockSpec(w1aug.shape, fixed),
            pl.BlockSpec(b1r.shape, fixed),
            pl.BlockSpec(w2aug.shape, fixed),
            pl.BlockSpec(b2r.shape, fixed),
            pl.BlockSpec(fwb.shape, fixed),
            pl.BlockSpec(fbr.shape, fixed),
        ],
        out_specs=pl.BlockSpec((TR, out_dim), lambda v, c: (c, v)),
        out_shape=jax.ShapeDtypeStruct((B, V * out_dim), jnp.float32),
        compiler_params=pltpu.CompilerParams(
            vmem_limit_bytes=100 * 1024 * 1024),
    )(X, mpn, view_W, bv, w1aug, b1r, w2aug, b2r, fwb, fbr)
    return out2d.reshape(B, V, out_dim)
